# baseline (speedup 1.0000x reference)
import functools

import jax
import jax.numpy as jnp
from jax import lax
from jax.experimental import pallas as pl
from jax.experimental.pallas import tpu as pltpu
from jax.experimental.pallas import tpu_sc as plsc

D_MODEL = 1024


@functools.lru_cache(maxsize=None)
def _build_emb(B: int, D: int, V: int):
    info = plsc.get_sparse_core_info()
    NC, NS = info.num_cores, info.num_subcores
    NW = NC * NS
    assert B % NW == 0
    bpw = B // NW
    C = 64
    while bpw % C:
        C //= 2
    nchunk = bpw // C
    mesh = plsc.VectorSubcoreMesh(core_axis_name="c", subcore_axis_name="s")

    @functools.partial(
        pl.kernel,
        mesh=mesh,
        out_type=jax.ShapeDtypeStruct((B, D), jnp.float32),
        scratch_types=[
            pltpu.VMEM((bpw,), jnp.int32),
            pltpu.VMEM((C, D), jnp.float32),
            pltpu.SemaphoreType.DMA,
        ],
    )
    def emb(idx_hbm, table_hbm, out_hbm, idx_v, rows_v, sem):
        wid = lax.axis_index("s") * NC + lax.axis_index("c")
        base = wid * bpw
        pltpu.sync_copy(idx_hbm.at[pl.ds(base, bpw)], idx_v)

        def body(i, carry):
            pltpu.async_copy(
                table_hbm.at[idx_v.at[pl.ds(i * C, C)]], rows_v, sem
            ).wait()
            pltpu.sync_copy(rows_v, out_hbm.at[pl.ds(base + i * C, C)])
            return carry

        lax.fori_loop(0, nchunk, body, 0)

    return emb


def kernel(seqs, W_embed):
    batch, seq = seqs.shape
    V, D = W_embed.shape
    idx = seqs.reshape(-1).astype(jnp.int32)
    emb = _build_emb(batch * seq, D, V)
    out = emb(idx, W_embed)
    return out.reshape(batch, seq, D)
